# baseline (speedup 1.0000x reference)
import functools

import jax
import jax.numpy as jnp
from jax import lax
from jax.experimental import pallas as pl
from jax.experimental.pallas import tpu as pltpu
from jax.experimental.pallas import tpu_sc as plsc

FEAT_DIM = 80
D_MODEL = 256
PH_DIM = 32
SP_DIM = 8
E_ROW = 64
B = 32
T = 1500
TB = 768
T_PAD = 2 * TB

_NC = 2
_NS = 16
_NW = _NC * _NS

_ROUND = 384


def _sc_gather(idx_flat, table_flat):
    mesh = plsc.VectorSubcoreMesh(core_axis_name="c", subcore_axis_name="s")

    @functools.partial(
        pl.kernel,
        mesh=mesh,
        out_type=jax.ShapeDtypeStruct((B, TB * 2 * E_ROW), jnp.float32),
        compiler_params=pltpu.CompilerParams(use_tc_tiling_on_sc=False,
                                             needs_layout_passes=False),
        scratch_types=[
            pltpu.VMEM((T_PAD,), jnp.int32),
            pltpu.VMEM((256 * E_ROW,), jnp.float32),
            pltpu.VMEM((TB * 2 * E_ROW,), jnp.float32),
            pltpu.SemaphoreType.DMA,
        ],
    )
    def gather_kernel(idx_hbm, tab_hbm, out_hbm, idx_v, tab_v, packed, wsem):
        b = lax.axis_index("s") * _NC + lax.axis_index("c")
        pltpu.sync_copy(idx_hbm.at[pl.ds(b * T_PAD, T_PAD)], idx_v)
        pltpu.sync_copy(tab_hbm, tab_v)
        iota = lax.iota(jnp.int32, 16)
        nchunk = 4
        grp = TB // (16 * nchunk)
        writes = []
        for chunk in range(nchunk):

            def body(g, carry):
                gg = chunk * grp + g
                ra = idx_v[pl.ds(gg * 16, 16)] * E_ROW
                rb = idx_v[pl.ds(TB + gg * 16, 16)] * E_ROW
                sbase = (gg * 16 + iota) * (2 * E_ROW)
                for l in range(E_ROW):
                    plsc.store_scatter(packed, [sbase + l],
                                       plsc.load_gather(tab_v, [ra + l]))
                    plsc.store_scatter(packed, [sbase + E_ROW + l],
                                       plsc.load_gather(tab_v, [rb + l]))
                return carry

            lax.fori_loop(0, grp, body, 0)
            n = grp * 16 * 2 * E_ROW
            writes.append(
                pltpu.async_copy(
                    packed.at[pl.ds(chunk * n, n)],
                    out_hbm.at[b, pl.ds(chunk * n, n)],
                    wsem,
                )
            )
        for w in writes:
            w.wait()

    return gather_kernel(idx_flat, table_flat)


def _tc_body(aft_ref, ptt_ref, e_ref, wa_ref, wpt_ref, wep_ref, b1_ref, g_ref, bt_ref, out_ref):
    k = pl.program_id(1)
    cdims = (((0,), (0,)), ((), ()))
    vs = []
    for j in range(8):
        aft = aft_ref[j, :, pl.ds(k * TB, TB)]
        h = lax.dot_general(aft, wa_ref[...], cdims,
                            preferred_element_type=jnp.float32)
        h = h + jnp.dot(e_ref[j], wep_ref[0], preferred_element_type=jnp.float32)
        h = h + lax.dot_general(ptt_ref[j, :, pl.ds(k * TB, TB)], wpt_ref[...],
                                cdims, preferred_element_type=jnp.float32)
        h = h + b1_ref[...]
        s1 = jnp.sum(h, axis=1, keepdims=True)
        s2 = jnp.sum(h * h, axis=1, keepdims=True)
        mu = s1 * (1.0 / D_MODEL)
        var = s2 * (1.0 / D_MODEL) - mu * mu
        rs = lax.rsqrt(var + 1e-5)
        hn = (h * rs - mu * rs) * g_ref[...] + bt_ref[...]
        vs.append(0.5 * hn * (1.0 + lax.erf(hn * 0.7071067811865476)))
    out_ref[...] = jnp.stack(vs, axis=1)


def kernel(af, pi, pt, si, phoneme_table, speaker_table, W1, b1, gamma, beta):
    idx = pi.astype(jnp.int32) * 2 + si.astype(jnp.int32)
    idx_flat = jnp.pad(idx, ((0, 0), (0, T_PAD - T))).reshape(-1)

    comb = jnp.concatenate(
        [
            jnp.repeat(phoneme_table, 2, axis=0),
            jnp.tile(speaker_table, (128, 1)),
            jnp.zeros((256, E_ROW - PH_DIM - SP_DIM), jnp.float32),
        ],
        axis=1,
    )

    e = _sc_gather(idx_flat, comb.reshape(-1)).reshape(B, TB, 2 * E_ROW)

    we = W1[FEAT_DIM + 1:]
    wep = jnp.stack(
        [
            jnp.concatenate([we, jnp.zeros((2 * E_ROW - PH_DIM - SP_DIM, D_MODEL),
                                           jnp.float32)]),
            jnp.concatenate([jnp.zeros((E_ROW, D_MODEL), jnp.float32), we,
                             jnp.zeros((E_ROW - PH_DIM - SP_DIM, D_MODEL),
                                       jnp.float32)]),
        ]
    )

    wa = W1[:FEAT_DIM]
    wpt = W1[FEAT_DIM:FEAT_DIM + 1]

    out_l = pl.pallas_call(
        _tc_body,
        grid=(B // 8, (T + TB - 1) // TB),
        in_specs=[
            pl.BlockSpec((8, FEAT_DIM, T), lambda i, k: (i, 0, 0)),
            pl.BlockSpec((8, 1, T), lambda i, k: (i, 0, 0)),
            pl.BlockSpec((8, TB, 2 * E_ROW), lambda i, k: (i, 0, 0)),
            pl.BlockSpec((FEAT_DIM, D_MODEL), lambda i, k: (0, 0)),
            pl.BlockSpec((1, D_MODEL), lambda i, k: (0, 0)),
            pl.BlockSpec((1, 2 * E_ROW, D_MODEL), lambda i, k: (k, 0, 0)),
            pl.BlockSpec((1, D_MODEL), lambda i, k: (0, 0)),
            pl.BlockSpec((1, D_MODEL), lambda i, k: (0, 0)),
            pl.BlockSpec((1, D_MODEL), lambda i, k: (0, 0)),
        ],
        out_specs=pl.BlockSpec((TB, 8, D_MODEL), lambda i, k: (k, i, 0)),
        out_shape=jax.ShapeDtypeStruct((T, B, D_MODEL), jnp.float32),
        compiler_params=pltpu.CompilerParams(vmem_limit_bytes=100 * 1024 * 1024),
    )(af.transpose(0, 2, 1), pt.transpose(0, 2, 1), e, wa, wpt, wep,
      b1.reshape(1, D_MODEL), gamma.reshape(1, D_MODEL), beta.reshape(1, D_MODEL))

    return out_l.transpose(1, 0, 2)


# speedup vs baseline: 1.5040x; 1.5040x over previous
import functools

import jax
import jax.numpy as jnp
from jax import lax
from jax.experimental import pallas as pl
from jax.experimental.pallas import tpu as pltpu
from jax.experimental.pallas import tpu_sc as plsc

FEAT_DIM = 80
D_MODEL = 256
PH_DIM = 32
SP_DIM = 8
E_DIM = 128
B = 32
T = 1500
T_PAD = 1504

_NC = 2
_NS = 16
_NW = _NC * _NS

TB = 768

_ROUNDS = ((0, 384), (384, 384), (768, 384), (1152, 348))
_RMAX = 384


def _sc_gather(idx_flat, table):
    mesh = plsc.VectorSubcoreMesh(core_axis_name="c", subcore_axis_name="s")

    @functools.partial(
        pl.kernel,
        mesh=mesh,
        out_type=jax.ShapeDtypeStruct((B, T, E_DIM), jnp.float32),
        scratch_types=[
            pltpu.VMEM((T_PAD,), jnp.int32),
            pltpu.VMEM((2, _RMAX, E_DIM), jnp.float32),
            pltpu.SemaphoreType.DMA,
            pltpu.SemaphoreType.DMA,
        ],
    )
    def gather_kernel(idx_hbm, tab_hbm, out_hbm, idx_v, rows_v, gsem, wsem):
        b = lax.axis_index("s") * _NC + lax.axis_index("c")
        pltpu.sync_copy(idx_hbm.at[pl.ds(b * T_PAD, T_PAD)], idx_v)
        writes = []
        for r, (start, n) in enumerate(_ROUNDS):
            buf = rows_v.at[r % 2]
            gathers = []
            off = 0
            while off < n:
                c = min(128, n - off)
                gathers.append(
                    pltpu.async_copy(
                        tab_hbm.at[idx_v.at[pl.ds(start + off, c)]],
                        buf.at[pl.ds(off, c)],
                        gsem,
                    )
                )
                off += c
            if r >= 2:
                writes[r - 2].wait()
            for g in gathers:
                g.wait()
            writes.append(
                pltpu.async_copy(
                    buf.at[pl.ds(0, n)], out_hbm.at[b, pl.ds(start, n)], wsem
                )
            )
        for w in writes[-2:]:
            w.wait()

    return gather_kernel(idx_flat, table)


def _tc_body(aft_ref, ptt_ref, e_ref, wa_ref, wpt_ref, we_ref, b1_ref, g_ref, bt_ref, out_ref):
    cdims = (((0,), (0,)), ((), ()))
    vs = []
    for j in range(8):
        h = lax.dot_general(aft_ref[j], wa_ref[...], cdims,
                            preferred_element_type=jnp.float32)
        h = h + jnp.dot(e_ref[j], we_ref[...], preferred_element_type=jnp.float32)
        h = h + lax.dot_general(ptt_ref[j], wpt_ref[...], cdims,
                                preferred_element_type=jnp.float32)
        h = h + b1_ref[...]
        mu = jnp.mean(h, axis=1, keepdims=True)
        var = jnp.mean((h - mu) ** 2, axis=1, keepdims=True)
        hn = (h - mu) * lax.rsqrt(var + 1e-5)
        hn = hn * g_ref[...] + bt_ref[...]
        vs.append(0.5 * hn * (1.0 + lax.erf(hn * 0.7071067811865476)))
    out_ref[...] = jnp.stack(vs, axis=1)


def kernel(af, pi, pt, si, phoneme_table, speaker_table, W1, b1, gamma, beta):
    idx = pi.astype(jnp.int32) * 2 + si.astype(jnp.int32)
    idx_flat = jnp.pad(idx, ((0, 0), (0, T_PAD - T))).reshape(-1)

    comb = jnp.concatenate(
        [
            jnp.repeat(phoneme_table, 2, axis=0),
            jnp.tile(speaker_table, (128, 1)),
            jnp.zeros((256, E_DIM - PH_DIM - SP_DIM), jnp.float32),
        ],
        axis=1,
    )

    e = _sc_gather(idx_flat, comb)

    wa = W1[:FEAT_DIM]
    wpt = W1[FEAT_DIM:FEAT_DIM + 1]
    we = jnp.concatenate(
        [W1[FEAT_DIM + 1:], jnp.zeros((E_DIM - PH_DIM - SP_DIM, D_MODEL), jnp.float32)]
    )

    out_l = pl.pallas_call(
        _tc_body,
        grid=(B // 8,),
        in_specs=[
            pl.BlockSpec((8, FEAT_DIM, T), lambda i: (i, 0, 0)),
            pl.BlockSpec((8, 1, T), lambda i: (i, 0, 0)),
            pl.BlockSpec((8, T, E_DIM), lambda i: (i, 0, 0)),
            pl.BlockSpec((FEAT_DIM, D_MODEL), lambda i: (0, 0)),
            pl.BlockSpec((1, D_MODEL), lambda i: (0, 0)),
            pl.BlockSpec((E_DIM, D_MODEL), lambda i: (0, 0)),
            pl.BlockSpec((1, D_MODEL), lambda i: (0, 0)),
            pl.BlockSpec((1, D_MODEL), lambda i: (0, 0)),
            pl.BlockSpec((1, D_MODEL), lambda i: (0, 0)),
        ],
        out_specs=pl.BlockSpec((T, 8, D_MODEL), lambda i: (0, i, 0)),
        out_shape=jax.ShapeDtypeStruct((T, B, D_MODEL), jnp.float32),
        compiler_params=pltpu.CompilerParams(vmem_limit_bytes=100 * 1024 * 1024),
    )(af.transpose(0, 2, 1), pt.transpose(0, 2, 1), e, wa, wpt, we,
      b1.reshape(1, D_MODEL), gamma.reshape(1, D_MODEL), beta.reshape(1, D_MODEL))

    return out_l.transpose(1, 0, 2)
